# baseline (speedup 1.0000x reference)
import functools

import jax
import jax.numpy as jnp
from jax.experimental import pallas as pl
from jax.experimental.pallas import tpu as pltpu

SEQ_CHUNK = 64


def _lstm_body(x_ref, wih_ref, whh_ref, b_ref, op_ref, out_ref,
               xw_ref, h_ref, c_ref, top_ref):
    i = pl.program_id(0)
    chunk = x_ref.shape[0]
    batch = x_ref.shape[1]
    hid = whh_ref.shape[0]

    @pl.when(i == 0)
    def _init():
        h_ref[...] = jnp.zeros_like(h_ref)
        c_ref[...] = jnp.zeros_like(c_ref)
        top_ref[...] = jnp.zeros_like(top_ref)

    x2d = x_ref[...].reshape(chunk * batch, x_ref.shape[2])
    xw_ref[...] = jax.lax.dot_general(
        x2d.astype(jnp.bfloat16), wih_ref[...], (((1,), (0,)), ((), ())),
        preferred_element_type=jnp.float32) + b_ref[0:1, :]

    t0 = i * chunk
    whh_bf = whh_ref[...]

    G = 2
    gb = batch // G
    opv = [op_ref[pl.ds(g * gb, gb), :] for g in range(G)]

    def step(j, carry):
        hs, cs, tops = carry
        nh, nc, nt = [], [], []
        for g in range(G):
            gates = xw_ref[pl.ds(j * batch + g * gb, gb), :] + \
                jax.lax.dot_general(
                    hs[g].astype(jnp.bfloat16), whh_bf,
                    (((1,), (0,)), ((), ())),
                    preferred_element_type=jnp.float32)
            ig = jax.nn.sigmoid(gates[:, :hid])
            fg = jax.nn.sigmoid(gates[:, hid:2 * hid])
            gg = jnp.tanh(gates[:, 2 * hid:3 * hid])
            og = jax.nn.sigmoid(gates[:, 3 * hid:])
            c = fg * cs[g] + ig * gg
            h = og * jnp.tanh(c)
            nt.append(jnp.where(opv[g] == t0 + j, h, tops[g]))
            nh.append(h)
            nc.append(c)
        return tuple(nh), tuple(nc), tuple(nt)

    init = (tuple(h_ref[pl.ds(g * gb, gb), :] for g in range(G)),
            tuple(c_ref[pl.ds(g * gb, gb), :] for g in range(G)),
            tuple(top_ref[pl.ds(g * gb, gb), :] for g in range(G)))
    hs, cs, tops = jax.lax.fori_loop(0, chunk, step, init, unroll=4)
    for g in range(G):
        h_ref[pl.ds(g * gb, gb), :] = hs[g]
        c_ref[pl.ds(g * gb, gb), :] = cs[g]
        top_ref[pl.ds(g * gb, gb), :] = tops[g]
        out_ref[pl.ds(g * gb, gb), :] = tops[g]


@jax.jit
def _run(x, wih_t, whh_t, b2, op_b):
    seq, batch, d = x.shape
    hid = whh_t.shape[0]
    grid = (seq // SEQ_CHUNK,)
    return pl.pallas_call(
        _lstm_body,
        grid=grid,
        in_specs=[
            pl.BlockSpec((SEQ_CHUNK, batch, d), lambda i: (i, 0, 0)),
            pl.BlockSpec((d, 4 * hid), lambda i: (0, 0)),
            pl.BlockSpec((hid, 4 * hid), lambda i: (0, 0)),
            pl.BlockSpec((8, 4 * hid), lambda i: (0, 0)),
            pl.BlockSpec((batch, hid), lambda i: (0, 0)),
        ],
        out_specs=pl.BlockSpec((batch, hid), lambda i: (0, 0)),
        out_shape=jax.ShapeDtypeStruct((batch, hid), jnp.float32),
        scratch_shapes=[
            pltpu.VMEM((SEQ_CHUNK * batch, 4 * hid), jnp.float32),
            pltpu.VMEM((batch, hid), jnp.float32),
            pltpu.VMEM((batch, hid), jnp.float32),
            pltpu.VMEM((batch, hid), jnp.float32),
        ],
    )(x, wih_t, whh_t, b2, op_b)


def kernel(inputs, W_ih, W_hh, b_ih, b_hh, op):
    seq, batch, d = inputs.shape
    hid = W_hh.shape[1]
    wih_t = W_ih.T.astype(jnp.bfloat16)
    whh_t = W_hh.T.astype(jnp.bfloat16)
    b2 = jnp.broadcast_to((b_ih + b_hh)[None, :], (8, 4 * hid))
    op_b = jnp.broadcast_to(op.astype(jnp.int32)[:, None], (batch, hid))
    return _run(inputs, wih_t, whh_t, b2, op_b)


# speedup vs baseline: 1.0889x; 1.0889x over previous
import functools

import jax
import jax.numpy as jnp
from jax.experimental import pallas as pl
from jax.experimental.pallas import tpu as pltpu

SEQ_CHUNK = 64


def _lstm_body(x_ref, wih_ref, whh_ref, b_ref, op_ref, out_ref,
               xw_ref, h_ref, c_ref, top_ref):
    i = pl.program_id(0)
    chunk = x_ref.shape[0]
    batch = x_ref.shape[1]
    hid = whh_ref.shape[0]

    @pl.when(i == 0)
    def _init():
        h_ref[...] = jnp.zeros_like(h_ref)
        c_ref[...] = jnp.zeros_like(c_ref)
        top_ref[...] = jnp.zeros_like(top_ref)

    x2d = x_ref[...].reshape(chunk * batch, x_ref.shape[2])
    xw_ref[...] = jax.lax.dot_general(
        x2d.astype(jnp.bfloat16), wih_ref[...], (((1,), (0,)), ((), ())),
        preferred_element_type=jnp.float32) + b_ref[0:1, :]

    opv = op_ref[...]
    t0 = i * chunk
    whh_bf = whh_ref[...]

    def step(j, carry):
        h, c, top = carry
        gates = xw_ref[pl.ds(j * batch, batch), :] + jax.lax.dot_general(
            h.astype(jnp.bfloat16), whh_bf, (((1,), (0,)), ((), ())),
            preferred_element_type=jnp.float32)
        ig = jax.nn.sigmoid(gates[:, :hid])
        fg = jax.nn.sigmoid(gates[:, hid:2 * hid])
        gg = jnp.tanh(gates[:, 2 * hid:3 * hid])
        og = jax.nn.sigmoid(gates[:, 3 * hid:])
        c = fg * c + ig * gg
        h = og * jnp.tanh(c)
        top = jnp.where(opv == t0 + j, h, top)
        return h, c, top

    h, c, top = jax.lax.fori_loop(
        0, chunk, step, (h_ref[...], c_ref[...], top_ref[...]), unroll=8)
    h_ref[...] = h
    c_ref[...] = c
    top_ref[...] = top
    out_ref[...] = top


@jax.jit
def _run(x, wih_t, whh_t, b2, op_b):
    seq, batch, d = x.shape
    hid = whh_t.shape[0]
    grid = (seq // SEQ_CHUNK,)
    return pl.pallas_call(
        _lstm_body,
        grid=grid,
        in_specs=[
            pl.BlockSpec((SEQ_CHUNK, batch, d), lambda i: (i, 0, 0)),
            pl.BlockSpec((d, 4 * hid), lambda i: (0, 0)),
            pl.BlockSpec((hid, 4 * hid), lambda i: (0, 0)),
            pl.BlockSpec((8, 4 * hid), lambda i: (0, 0)),
            pl.BlockSpec((batch, hid), lambda i: (0, 0)),
        ],
        out_specs=pl.BlockSpec((batch, hid), lambda i: (0, 0)),
        out_shape=jax.ShapeDtypeStruct((batch, hid), jnp.float32),
        scratch_shapes=[
            pltpu.VMEM((SEQ_CHUNK * batch, 4 * hid), jnp.float32),
            pltpu.VMEM((batch, hid), jnp.float32),
            pltpu.VMEM((batch, hid), jnp.float32),
            pltpu.VMEM((batch, hid), jnp.float32),
        ],
    )(x, wih_t, whh_t, b2, op_b)


def kernel(inputs, W_ih, W_hh, b_ih, b_hh, op):
    seq, batch, d = inputs.shape
    hid = W_hh.shape[1]
    wih_t = W_ih.T.astype(jnp.bfloat16)
    whh_t = W_hh.T.astype(jnp.bfloat16)
    b2 = jnp.broadcast_to((b_ih + b_hh)[None, :], (8, 4 * hid))
    op_b = jnp.broadcast_to(op.astype(jnp.int32)[:, None], (batch, hid))
    return _run(inputs, wih_t, whh_t, b2, op_b)


# speedup vs baseline: 1.0935x; 1.0042x over previous
import functools

import jax
import jax.numpy as jnp
from jax.experimental import pallas as pl
from jax.experimental.pallas import tpu as pltpu

SEQ_CHUNK = 128


def _lstm_body(x_ref, wih_ref, whh_ref, b_ref, op_ref, out_ref,
               xw_ref, h_ref, c_ref, top_ref):
    i = pl.program_id(0)
    chunk = x_ref.shape[0]
    batch = x_ref.shape[1]
    hid = whh_ref.shape[0]

    @pl.when(i == 0)
    def _init():
        h_ref[...] = jnp.zeros_like(h_ref)
        c_ref[...] = jnp.zeros_like(c_ref)
        top_ref[...] = jnp.zeros_like(top_ref)

    x2d = x_ref[...].reshape(chunk * batch, x_ref.shape[2])
    xw_ref[...] = jax.lax.dot_general(
        x2d.astype(jnp.bfloat16), wih_ref[...], (((1,), (0,)), ((), ())),
        preferred_element_type=jnp.float32) + b_ref[0:1, :]

    opv = op_ref[...]
    t0 = i * chunk
    whh_bf = whh_ref[...]

    def step(j, carry):
        h, c, top = carry
        gates = xw_ref[pl.ds(j * batch, batch), :] + jax.lax.dot_general(
            h.astype(jnp.bfloat16), whh_bf, (((1,), (0,)), ((), ())),
            preferred_element_type=jnp.float32)
        ig = jax.nn.sigmoid(gates[:, :hid])
        fg = jax.nn.sigmoid(gates[:, hid:2 * hid])
        gg = jnp.tanh(gates[:, 2 * hid:3 * hid])
        og = jax.nn.sigmoid(gates[:, 3 * hid:])
        c = fg * c + ig * gg
        h = og * jnp.tanh(c)
        top = jnp.where(opv == t0 + j, h, top)
        return h, c, top

    h, c, top = jax.lax.fori_loop(
        0, chunk, step, (h_ref[...], c_ref[...], top_ref[...]), unroll=8)
    h_ref[...] = h
    c_ref[...] = c
    top_ref[...] = top
    out_ref[...] = top


@jax.jit
def _run(x, wih_t, whh_t, b2, op_b):
    seq, batch, d = x.shape
    hid = whh_t.shape[0]
    grid = (seq // SEQ_CHUNK,)
    return pl.pallas_call(
        _lstm_body,
        grid=grid,
        in_specs=[
            pl.BlockSpec((SEQ_CHUNK, batch, d), lambda i: (i, 0, 0)),
            pl.BlockSpec((d, 4 * hid), lambda i: (0, 0)),
            pl.BlockSpec((hid, 4 * hid), lambda i: (0, 0)),
            pl.BlockSpec((8, 4 * hid), lambda i: (0, 0)),
            pl.BlockSpec((batch, hid), lambda i: (0, 0)),
        ],
        out_specs=pl.BlockSpec((batch, hid), lambda i: (0, 0)),
        out_shape=jax.ShapeDtypeStruct((batch, hid), jnp.float32),
        scratch_shapes=[
            pltpu.VMEM((SEQ_CHUNK * batch, 4 * hid), jnp.float32),
            pltpu.VMEM((batch, hid), jnp.float32),
            pltpu.VMEM((batch, hid), jnp.float32),
            pltpu.VMEM((batch, hid), jnp.float32),
        ],
    )(x, wih_t, whh_t, b2, op_b)


def kernel(inputs, W_ih, W_hh, b_ih, b_hh, op):
    seq, batch, d = inputs.shape
    hid = W_hh.shape[1]
    wih_t = W_ih.T.astype(jnp.bfloat16)
    whh_t = W_hh.T.astype(jnp.bfloat16)
    b2 = jnp.broadcast_to((b_ih + b_hh)[None, :], (8, 4 * hid))
    op_b = jnp.broadcast_to(op.astype(jnp.int32)[:, None], (batch, hid))
    return _run(inputs, wih_t, whh_t, b2, op_b)


# speedup vs baseline: 1.0998x; 1.0058x over previous
import functools

import jax
import jax.numpy as jnp
from jax.experimental import pallas as pl
from jax.experimental.pallas import tpu as pltpu

SEQ_CHUNK = 128


def _lstm_body(x_ref, wih_ref, whh_ref, b_ref, op_ref, out_ref,
               xw_ref, h_ref, c_ref, top_ref):
    i = pl.program_id(0)
    chunk = x_ref.shape[0]
    batch = x_ref.shape[1]
    hid = whh_ref.shape[0]

    @pl.when(i == 0)
    def _init():
        h_ref[...] = jnp.zeros_like(h_ref)
        c_ref[...] = jnp.zeros_like(c_ref)
        top_ref[...] = jnp.zeros_like(top_ref)

    x2d = x_ref[...].reshape(chunk * batch, x_ref.shape[2])
    xw_ref[...] = jax.lax.dot_general(
        x2d.astype(jnp.bfloat16), wih_ref[...], (((1,), (0,)), ((), ())),
        preferred_element_type=jnp.float32)

    opv = op_ref[...]
    t0 = i * chunk
    whh_bf = whh_ref[...]
    brow = b_ref[0:1, :]

    def sig(v):
        return 0.5 * jnp.tanh(0.5 * v) + 0.5

    def step(j, carry):
        h, c, top = carry
        gates = (xw_ref[pl.ds(j * batch, batch), :] + brow) + \
            jax.lax.dot_general(
                h.astype(jnp.bfloat16), whh_bf, (((1,), (0,)), ((), ())),
                preferred_element_type=jnp.float32)
        ig = sig(gates[:, :hid])
        fg = sig(gates[:, hid:2 * hid])
        gg = jnp.tanh(gates[:, 2 * hid:3 * hid])
        og = sig(gates[:, 3 * hid:])
        c = fg * c + ig * gg
        h = og * jnp.tanh(c)
        top = jnp.where(opv == t0 + j, h, top)
        return h, c, top

    h, c, top = jax.lax.fori_loop(
        0, chunk, step, (h_ref[...], c_ref[...], top_ref[...]), unroll=8)
    h_ref[...] = h
    c_ref[...] = c
    top_ref[...] = top
    out_ref[...] = top


@jax.jit
def _run(x, wih_t, whh_t, b2, op_b):
    seq, batch, d = x.shape
    hid = whh_t.shape[0]
    grid = (seq // SEQ_CHUNK,)
    return pl.pallas_call(
        _lstm_body,
        grid=grid,
        in_specs=[
            pl.BlockSpec((SEQ_CHUNK, batch, d), lambda i: (i, 0, 0)),
            pl.BlockSpec((d, 4 * hid), lambda i: (0, 0)),
            pl.BlockSpec((hid, 4 * hid), lambda i: (0, 0)),
            pl.BlockSpec((8, 4 * hid), lambda i: (0, 0)),
            pl.BlockSpec((batch, hid), lambda i: (0, 0)),
        ],
        out_specs=pl.BlockSpec((batch, hid), lambda i: (0, 0)),
        out_shape=jax.ShapeDtypeStruct((batch, hid), jnp.float32),
        scratch_shapes=[
            pltpu.VMEM((SEQ_CHUNK * batch, 4 * hid), jnp.float32),
            pltpu.VMEM((batch, hid), jnp.float32),
            pltpu.VMEM((batch, hid), jnp.float32),
            pltpu.VMEM((batch, hid), jnp.float32),
        ],
    )(x, wih_t, whh_t, b2, op_b)


def kernel(inputs, W_ih, W_hh, b_ih, b_hh, op):
    seq, batch, d = inputs.shape
    hid = W_hh.shape[1]
    wih_t = W_ih.T.astype(jnp.bfloat16)
    whh_t = W_hh.T.astype(jnp.bfloat16)
    b2 = jnp.broadcast_to((b_ih + b_hh)[None, :], (8, 4 * hid))
    op_b = jnp.broadcast_to(op.astype(jnp.int32)[:, None], (batch, hid))
    return _run(inputs, wih_t, whh_t, b2, op_b)
